# Initial kernel scaffold; baseline (speedup 1.0000x reference)
import jax
import jax.numpy as jnp
from jax.experimental import pallas as pl


def kernel(ht, pos_t, same_scene_mask):
    raise NotImplementedError("write your pallas kernel here")


# speedup vs baseline: 94.9293x; 94.9293x over previous
import functools

import jax
import jax.numpy as jnp
from jax import lax
from jax.experimental import pallas as pl
from jax.experimental.pallas import tpu as pltpu
from jax.experimental.pallas import tpu_sc as plsc

N = 8192
H = 128
GRID = 4
G = GRID * GRID
AREA_SPAN = 2.0
HALF = AREA_SPAN / 2.0
EPS = 0.01
THR = HALF - EPS
INV_CELL = GRID / AREA_SPAN

NW = 32
IPW = N // NW
SB = 16
NSB = IPW // SB
JC = 256
LANES = 16
HC = H // LANES


def _sc_body(ht_hbm, px_hbm, py_hbm, rs_hbm, re_hbm, out_hbm,
             htc, pxc, pyc, rsi, rei, pxi, pyi, acc):
    cid = lax.axis_index("c")
    sid = lax.axis_index("s")
    wid = sid * 2 + cid
    i_base = wid * IPW
    iota = lax.iota(jnp.int32, LANES)

    def subblock(b, _):
        i0 = i_base + b * SB
        pltpu.sync_copy(rs_hbm.at[pl.ds(i0, SB)], rsi.at[pl.ds(0, SB)])
        pltpu.sync_copy(re_hbm.at[pl.ds(i0, SB)], rei.at[pl.ds(0, SB)])
        pltpu.sync_copy(px_hbm.at[pl.ds(i0, SB)], pxi.at[pl.ds(0, SB)])
        pltpu.sync_copy(py_hbm.at[pl.ds(i0, SB)], pyi.at[pl.ds(0, SB)])

        def zbody(k, _):
            acc[pl.ds(k * LANES, LANES)] = jnp.zeros((LANES,), jnp.float32)
            return 0

        lax.fori_loop(0, SB * G * H // LANES, zbody, 0)

        rs0 = rsi[pl.ds(0, LANES)][0]
        re_last = rei[pl.ds(SB - 1, LANES)][0]
        jb0 = (rs0 // 8) * 8
        nch = (re_last - jb0 + JC - 1) // JC

        def chunk(ci, _):
            jb = jb0 + ci * JC
            pltpu.sync_copy(ht_hbm.at[pl.ds(jb * H, JC * H)], htc)
            pltpu.sync_copy(px_hbm.at[pl.ds(jb, JC)], pxc)
            pltpu.sync_copy(py_hbm.at[pl.ds(jb, JC)], pyc)

            def per_i(s, _):
                rs = rsi[pl.ds(s, LANES)][0]
                ren = rei[pl.ds(s, LANES)][0]
                xi = pxi[pl.ds(s, LANES)][0]
                yi = pyi[pl.ds(s, LANES)][0]
                iglob = i0 + s
                lo = jnp.clip(rs - jb, 0, JC)
                hi = jnp.clip(ren - jb, lo, JC)
                glo = lo // LANES
                ghi = jnp.maximum((hi + LANES - 1) // LANES, glo)
                abase = s * (G * H)

                def per_jg(gidx, _):
                    jb16 = gidx * LANES
                    xv = pxc[pl.ds(jb16, LANES)]
                    yv = pyc[pl.ds(jb16, LANES)]
                    jg = (jb + jb16) + iota
                    relx = xv - xi
                    rely = yv - yi
                    okv = ((relx < THR) & (relx > -THR)
                           & (rely < THR) & (rely > -THR)
                           & (jg != iglob) & (jg >= rs) & (jg < ren))
                    gxv = ((relx + HALF) * INV_CELL).astype(jnp.int32)
                    gyv = ((rely + HALF) * INV_CELL).astype(jnp.int32)
                    gv = jnp.where(okv, gxv * GRID + gyv, 0)
                    wv = jnp.where(okv, jnp.float32(1.0), jnp.float32(0.0))
                    offv = abase + gv * H
                    for l in range(LANES):
                        wl = wv[l]
                        ol = offv[l]
                        hb = (jb16 + l) * H
                        for c in range(HC):
                            v = htc[pl.ds(hb + c * LANES, LANES)] * wl
                            plsc.addupdate(
                                acc.at[pl.ds(ol + c * LANES, LANES)], v)
                    return 0

                lax.fori_loop(glo, ghi, per_jg, 0)
                return 0

            lax.fori_loop(0, SB, per_i, 0)
            return 0

        lax.fori_loop(0, nch, chunk, 0)
        pltpu.sync_copy(acc, out_hbm.at[pl.ds(i0 * G * H, SB * G * H)])
        return 0

    lax.fori_loop(0, NSB, subblock, 0)


_sc_pool = functools.partial(
    pl.kernel,
    out_type=jax.ShapeDtypeStruct((N * G * H,), jnp.float32),
    mesh=plsc.VectorSubcoreMesh(core_axis_name="c", subcore_axis_name="s"),
    scratch_types=[
        pltpu.VMEM((JC * H,), jnp.float32),
        pltpu.VMEM((JC,), jnp.float32),
        pltpu.VMEM((JC,), jnp.float32),
        pltpu.VMEM((2 * SB,), jnp.int32),
        pltpu.VMEM((2 * SB,), jnp.int32),
        pltpu.VMEM((2 * SB,), jnp.float32),
        pltpu.VMEM((2 * SB,), jnp.float32),
        pltpu.VMEM((SB * G * H,), jnp.float32),
    ],
)(_sc_body)


def kernel(ht, pos_t, same_scene_mask):
    ht2 = ht.reshape(N, H)
    pos = pos_t.reshape(N, 2)
    scene = same_scene_mask.reshape(N)
    rs = jnp.searchsorted(scene, scene, side="left").astype(jnp.int32)
    re_ = jnp.searchsorted(scene, scene, side="right").astype(jnp.int32)
    zf = jnp.zeros((JC,), jnp.float32)
    ht_pad = jnp.concatenate(
        [ht2, jnp.zeros((JC, H), ht2.dtype)], axis=0).reshape((N + JC) * H)
    px_pad = jnp.concatenate([pos[:, 0], zf])
    py_pad = jnp.concatenate([pos[:, 1], zf])
    out = _sc_pool(ht_pad, px_pad, py_pad, rs, re_)
    return out.reshape(N, G, H)


# speedup vs baseline: 178.0293x; 1.8754x over previous
import functools

import jax
import jax.numpy as jnp
from jax import lax
from jax.experimental import pallas as pl
from jax.experimental.pallas import tpu as pltpu
from jax.experimental.pallas import tpu_sc as plsc

N = 8192
H = 128
GRID = 4
G = GRID * GRID
AREA_SPAN = 2.0
HALF = AREA_SPAN / 2.0
EPS = 0.01
THR = HALF - EPS
INV_CELL = GRID / AREA_SPAN

NW = 32
IPW = N // NW
SB = 16
NSB = IPW // SB
JC = 256
LANES = 16
HC = H // LANES
ACC = SB * G * H


def _sc_body(ht_hbm, px_hbm, py_hbm, mb_hbm, mp_hbm, out_hbm,
             htc, pxc, pyc, metab, metap, acc, sem):
    cid = lax.axis_index("c")
    sid = lax.axis_index("s")
    wid = sid * 2 + cid
    i_base = wid * IPW
    iota = lax.iota(jnp.int32, LANES)
    zeros16 = jnp.zeros((LANES,), jnp.float32)

    def subblock(b, _):
        i0 = i_base + b * SB
        m1 = pltpu.async_copy(mb_hbm.at[pl.ds(i0 * 2, SB * 2)],
                              metab.at[pl.ds(0, SB * 2)], sem)
        m2 = pltpu.async_copy(mp_hbm.at[pl.ds(i0 * 2, SB * 2)],
                              metap.at[pl.ds(0, SB * 2)], sem)
        m1.wait()
        m2.wait()

        def zbody(k, _):
            for u in range(16):
                acc[pl.ds((k * 16 + u) * LANES, LANES)] = zeros16
            return 0

        lax.fori_loop(0, ACC // LANES // 16, zbody, 0)

        rs0 = metab[pl.ds(0, LANES)][0]
        re_last = metab[pl.ds(2 * (SB - 1) + 1, LANES)][0]
        jb0 = (rs0 // 8) * 8
        nch = (re_last - jb0 + JC - 1) // JC

        def chunk(ci, _):
            jb = jb0 + ci * JC
            c1 = pltpu.async_copy(ht_hbm.at[pl.ds(jb * H, JC * H)], htc, sem)
            c2 = pltpu.async_copy(px_hbm.at[pl.ds(jb, JC)], pxc, sem)
            c3 = pltpu.async_copy(py_hbm.at[pl.ds(jb, JC)], pyc, sem)
            c1.wait()
            c2.wait()
            c3.wait()

            def per_i(s, _):
                rs = metab[pl.ds(2 * s, LANES)][0]
                ren = metab[pl.ds(2 * s + 1, LANES)][0]
                xi = metap[pl.ds(2 * s, LANES)][0]
                yi = metap[pl.ds(2 * s + 1, LANES)][0]
                iglob = i0 + s
                lo = jnp.clip(rs - jb, 0, JC)
                hi = jnp.clip(ren - jb, lo, JC)
                glo = lo // LANES
                ghi = jnp.maximum((hi + LANES - 1) // LANES, glo)
                abase = s * (G * H)

                def per_jg(gidx, _):
                    jb16 = gidx * LANES
                    xv = pxc[pl.ds(jb16, LANES)]
                    yv = pyc[pl.ds(jb16, LANES)]
                    jg = (jb + jb16) + iota
                    relx = xv - xi
                    rely = yv - yi
                    okv = ((relx < THR) & (relx > -THR)
                           & (rely < THR) & (rely > -THR)
                           & (jg != iglob) & (jg >= rs) & (jg < ren))
                    gxv = ((relx + HALF) * INV_CELL).astype(jnp.int32)
                    gyv = ((rely + HALF) * INV_CELL).astype(jnp.int32)
                    offv = jnp.where(okv, abase + (gxv * GRID + gyv) * H,
                                     ACC)
                    for l in range(LANES):
                        ol = offv[l]
                        hb = (jb16 + l) * H
                        for c in range(HC):
                            plsc.addupdate(
                                acc.at[pl.ds(ol + c * LANES, LANES)],
                                htc[pl.ds(hb + c * LANES, LANES)])
                    return 0

                lax.fori_loop(glo, ghi, per_jg, 0)
                return 0

            lax.fori_loop(0, SB, per_i, 0)
            return 0

        lax.fori_loop(0, nch, chunk, 0)
        pltpu.sync_copy(acc.at[pl.ds(0, ACC)],
                        out_hbm.at[pl.ds(i0 * G * H, ACC)])
        return 0

    lax.fori_loop(0, NSB, subblock, 0)


_sc_pool = functools.partial(
    pl.kernel,
    out_type=jax.ShapeDtypeStruct((N * G * H,), jnp.float32),
    mesh=plsc.VectorSubcoreMesh(core_axis_name="c", subcore_axis_name="s"),
    scratch_types=[
        pltpu.VMEM((JC * H,), jnp.float32),
        pltpu.VMEM((JC,), jnp.float32),
        pltpu.VMEM((JC,), jnp.float32),
        pltpu.VMEM((SB * 2 + LANES,), jnp.int32),
        pltpu.VMEM((SB * 2 + LANES,), jnp.float32),
        pltpu.VMEM((ACC + H,), jnp.float32),
        pltpu.SemaphoreType.DMA,
    ],
)(_sc_body)


def kernel(ht, pos_t, same_scene_mask):
    ht2 = ht.reshape(N, H)
    pos = pos_t.reshape(N, 2)
    scene = same_scene_mask.reshape(N)
    idx = jnp.arange(N, dtype=jnp.int32)
    prev_ne = jnp.concatenate(
        [jnp.ones((1,), bool), scene[1:] != scene[:-1]])
    next_ne = jnp.concatenate(
        [scene[1:] != scene[:-1], jnp.ones((1,), bool)])
    rs = lax.associative_scan(jnp.maximum, jnp.where(prev_ne, idx, 0))
    re_ = lax.associative_scan(jnp.minimum, jnp.where(next_ne, idx + 1, N),
                               reverse=True)
    mb = jnp.stack([rs, re_], axis=1).reshape(-1)
    mp = pos.reshape(-1)
    zf = jnp.zeros((JC,), jnp.float32)
    ht_pad = jnp.concatenate(
        [ht2, jnp.zeros((JC, H), ht2.dtype)], axis=0).reshape((N + JC) * H)
    px_pad = jnp.concatenate([pos[:, 0], zf])
    py_pad = jnp.concatenate([pos[:, 1], zf])
    out = _sc_pool(ht_pad, px_pad, py_pad, mb, mp)
    return out.reshape(N, G, H)


# speedup vs baseline: 498.2118x; 2.7985x over previous
import functools

import jax
import jax.numpy as jnp
from jax import lax
from jax.experimental import pallas as pl
from jax.experimental.pallas import tpu as pltpu
from jax.experimental.pallas import tpu_sc as plsc

N = 8192
H = 128
GRID = 4
G = GRID * GRID
AREA_SPAN = 2.0
HALF = AREA_SPAN / 2.0
EPS = 0.01
THR = HALF - EPS
INV_CELL = GRID / AREA_SPAN

NW = 32
IPW = N // NW
SB = 16
NSB = IPW // SB
JC = 256
LANES = 16
HC = H // LANES
ACC = SB * G * H


def _sc_body(ht_hbm, px_hbm, py_hbm, rs_hbm, re_hbm, out_hbm,
             htc, pxc, pyc, rsi, rei, pxi, pyi, acc, sem):
    cid = lax.axis_index("c")
    sid = lax.axis_index("s")
    wid = sid * 2 + cid
    i_base = wid * IPW
    iota = lax.iota(jnp.int32, LANES)
    ioff = iota * (G * H)
    doff = ACC + iota * H
    zeros16 = jnp.zeros((LANES,), jnp.float32)

    def subblock(b, _):
        i0 = i_base + b * SB
        m1 = pltpu.async_copy(rs_hbm.at[pl.ds(i0, SB)], rsi, sem)
        m2 = pltpu.async_copy(re_hbm.at[pl.ds(i0, SB)], rei, sem)
        m3 = pltpu.async_copy(px_hbm.at[pl.ds(i0, SB)], pxi, sem)
        m4 = pltpu.async_copy(py_hbm.at[pl.ds(i0, SB)], pyi, sem)
        m1.wait()
        m2.wait()
        m3.wait()
        m4.wait()

        def zbody(k, _):
            for u in range(16):
                acc[pl.ds((k * 16 + u) * LANES, LANES)] = zeros16
            return 0

        lax.fori_loop(0, ACC // LANES // 16, zbody, 0)

        rsv = rsi[pl.ds(0, LANES)]
        rev = rei[pl.ds(0, LANES)]
        xiv = pxi[pl.ds(0, LANES)]
        yiv = pyi[pl.ds(0, LANES)]
        iiv = i0 + iota
        rs0 = rsv[0]
        re_last = rev[LANES - 1]
        jb0 = (rs0 // 8) * 8
        nch = (re_last - jb0 + JC - 1) // JC

        def chunk(ci, _):
            jb = jb0 + ci * JC
            c1 = pltpu.async_copy(ht_hbm.at[pl.ds(jb * H, JC * H)], htc, sem)
            c2 = pltpu.async_copy(px_hbm.at[pl.ds(jb, JC)],
                                  pxc.at[pl.ds(0, JC)], sem)
            c3 = pltpu.async_copy(py_hbm.at[pl.ds(jb, JC)],
                                  pyc.at[pl.ds(0, JC)], sem)
            c1.wait()
            c2.wait()
            c3.wait()
            lo = jnp.clip(rs0 - jb, 0, JC)
            hi = jnp.clip(re_last - jb, lo, JC)

            def per_j(jl, _):
                xj = pxc[pl.ds(jl, LANES)][0]
                yj = pyc[pl.ds(jl, LANES)][0]
                jg = jb + jl
                relx = xj - xiv
                rely = yj - yiv
                okv = ((relx < THR) & (relx > -THR)
                       & (rely < THR) & (rely > -THR)
                       & (jg != iiv) & (jg >= rsv) & (jg < rev))
                gxv = ((relx + HALF) * INV_CELL).astype(jnp.int32)
                gyv = ((rely + HALF) * INV_CELL).astype(jnp.int32)
                offv = jnp.where(okv, ioff + (gxv * GRID + gyv) * H, doff)
                hb = jl * H
                vs = [htc[pl.ds(hb + c * LANES, LANES)] for c in range(HC)]
                for l in range(LANES):
                    ol = offv[l]
                    for c in range(HC):
                        plsc.addupdate(
                            acc.at[pl.ds(ol + c * LANES, LANES)], vs[c])
                return 0

            lax.fori_loop(lo, hi, per_j, 0)
            return 0

        lax.fori_loop(0, nch, chunk, 0)
        pltpu.sync_copy(acc.at[pl.ds(0, ACC)],
                        out_hbm.at[pl.ds(i0 * G * H, ACC)])
        return 0

    lax.fori_loop(0, NSB, subblock, 0)


_sc_pool = functools.partial(
    pl.kernel,
    out_type=jax.ShapeDtypeStruct((N * G * H,), jnp.float32),
    mesh=plsc.VectorSubcoreMesh(core_axis_name="c", subcore_axis_name="s"),
    scratch_types=[
        pltpu.VMEM((JC * H,), jnp.float32),
        pltpu.VMEM((JC + LANES,), jnp.float32),
        pltpu.VMEM((JC + LANES,), jnp.float32),
        pltpu.VMEM((SB,), jnp.int32),
        pltpu.VMEM((SB,), jnp.int32),
        pltpu.VMEM((SB,), jnp.float32),
        pltpu.VMEM((SB,), jnp.float32),
        pltpu.VMEM((ACC + SB * H,), jnp.float32),
        pltpu.SemaphoreType.DMA,
    ],
)(_sc_body)


def kernel(ht, pos_t, same_scene_mask):
    ht2 = ht.reshape(N, H)
    pos = pos_t.reshape(N, 2)
    scene = same_scene_mask.reshape(N)
    idx = jnp.arange(N, dtype=jnp.int32)
    prev_ne = jnp.concatenate(
        [jnp.ones((1,), bool), scene[1:] != scene[:-1]])
    next_ne = jnp.concatenate(
        [scene[1:] != scene[:-1], jnp.ones((1,), bool)])
    rs = lax.associative_scan(jnp.maximum, jnp.where(prev_ne, idx, 0))
    re_ = lax.associative_scan(jnp.minimum, jnp.where(next_ne, idx + 1, N),
                               reverse=True)
    zf = jnp.zeros((JC,), jnp.float32)
    ht_pad = jnp.concatenate(
        [ht2, jnp.zeros((JC, H), ht2.dtype)], axis=0).reshape((N + JC) * H)
    px_pad = jnp.concatenate([pos[:, 0], zf])
    py_pad = jnp.concatenate([pos[:, 1], zf])
    out = _sc_pool(ht_pad, px_pad, py_pad, rs, re_)
    return out.reshape(N, G, H)


# speedup vs baseline: 582.8673x; 1.1699x over previous
import functools

import jax
import jax.numpy as jnp
from jax import lax
from jax.experimental import pallas as pl
from jax.experimental.pallas import tpu as pltpu
from jax.experimental.pallas import tpu_sc as plsc

N = 8192
H = 128
GRID = 4
G = GRID * GRID
AREA_SPAN = 2.0
HALF = AREA_SPAN / 2.0
EPS = 0.01
THR = HALF - EPS
INV_CELL = GRID / AREA_SPAN

NW = 32
IPW = N // NW
SB = 16
NSB = IPW // SB
JC = 256
LANES = 16
HC = H // LANES
ACC = SB * G * H


def _sc_body(ht_hbm, px_hbm, py_hbm, rs_hbm, re_hbm, out_hbm,
             htc, pxc, pyc, rsw, rew, pxw, pyw, acc0, acc1, sem,
             semw0, semw1):
    cid = lax.axis_index("c")
    sid = lax.axis_index("s")
    wid = sid * 2 + cid
    i_base = wid * IPW
    iota = lax.iota(jnp.int32, LANES)
    ioff = iota * (G * H)
    doff = ACC + iota * H
    zeros16 = jnp.zeros((LANES,), jnp.float32)
    accs = (acc0, acc1)
    semws = (semw0, semw1)

    m1 = pltpu.async_copy(rs_hbm.at[pl.ds(i_base, IPW)], rsw, sem)
    m2 = pltpu.async_copy(re_hbm.at[pl.ds(i_base, IPW)], rew, sem)
    m3 = pltpu.async_copy(px_hbm.at[pl.ds(i_base, IPW)], pxw, sem)
    m4 = pltpu.async_copy(py_hbm.at[pl.ds(i_base, IPW)], pyw, sem)
    m1.wait()
    m2.wait()
    m3.wait()
    m4.wait()

    def run_subblock(b, acc, semw, drain_prev):
        i0 = i_base + b * SB
        rsv = rsw[pl.ds(b * SB, LANES)]
        rev = rew[pl.ds(b * SB, LANES)]
        xiv = pxw[pl.ds(b * SB, LANES)]
        yiv = pyw[pl.ds(b * SB, LANES)]
        iiv = i0 + iota
        rs0 = rsv[0]
        re_last = rev[LANES - 1]
        jb0 = (rs0 // 8) * 8
        nch = (re_last - jb0 + JC - 1) // JC

        c1 = pltpu.async_copy(ht_hbm.at[pl.ds(jb0 * H, JC * H)], htc, sem)
        c2 = pltpu.async_copy(px_hbm.at[pl.ds(jb0, JC)],
                              pxc.at[pl.ds(0, JC)], sem)
        c3 = pltpu.async_copy(py_hbm.at[pl.ds(jb0, JC)],
                              pyc.at[pl.ds(0, JC)], sem)
        drain_prev()

        def zbody(k, _):
            for u in range(16):
                acc[pl.ds((k * 16 + u) * LANES, LANES)] = zeros16
            return 0

        lax.fori_loop(0, ACC // LANES // 16, zbody, 0)
        c1.wait()
        c2.wait()
        c3.wait()

        def compute_chunk(jb):
            lo = jnp.clip(rs0 - jb, 0, JC)
            hi = jnp.clip(re_last - jb, lo, JC)

            def per_j(jl, _):
                xj = pxc[pl.ds(jl, LANES)][0]
                yj = pyc[pl.ds(jl, LANES)][0]
                jg = jb + jl
                relx = xj - xiv
                rely = yj - yiv
                okv = ((relx < THR) & (relx > -THR)
                       & (rely < THR) & (rely > -THR)
                       & (jg != iiv) & (jg >= rsv) & (jg < rev))
                gxv = ((relx + HALF) * INV_CELL).astype(jnp.int32)
                gyv = ((rely + HALF) * INV_CELL).astype(jnp.int32)
                offv = jnp.where(okv, ioff + (gxv * GRID + gyv) * H, doff)
                hb = jl * H
                vs = [htc[pl.ds(hb + c * LANES, LANES)] for c in range(HC)]
                for l in range(LANES):
                    ol = offv[l]
                    for c in range(HC):
                        plsc.addupdate(
                            acc.at[pl.ds(ol + c * LANES, LANES)], vs[c])
                return 0

            lax.fori_loop(lo, hi, per_j, 0)

        compute_chunk(jb0)

        def chunk(ci, _):
            jb = jb0 + ci * JC
            d1 = pltpu.async_copy(ht_hbm.at[pl.ds(jb * H, JC * H)], htc, sem)
            d2 = pltpu.async_copy(px_hbm.at[pl.ds(jb, JC)],
                                  pxc.at[pl.ds(0, JC)], sem)
            d3 = pltpu.async_copy(py_hbm.at[pl.ds(jb, JC)],
                                  pyc.at[pl.ds(0, JC)], sem)
            d1.wait()
            d2.wait()
            d3.wait()
            compute_chunk(jb)
            return 0

        lax.fori_loop(1, nch, chunk, 0)
        pltpu.async_copy(acc.at[pl.ds(0, ACC)],
                         out_hbm.at[pl.ds(i0 * G * H, ACC)], semw)

    def wb_drain(b, acc, semw):
        i0p = i_base + b * SB
        pltpu.make_async_copy(acc.at[pl.ds(0, ACC)],
                              out_hbm.at[pl.ds(i0p * G * H, ACC)],
                              semw).wait()

    def subpair(bp, _):
        for par in range(2):
            b = bp * 2 + par
            acc = accs[par]
            semw = semws[par]

            def drain_prev(b=b, acc=acc, semw=semw):
                @pl.when(b >= 2)
                def _():
                    wb_drain(b - 2, acc, semw)

            run_subblock(b, acc, semw, drain_prev)
        return 0

    lax.fori_loop(0, NSB // 2, subpair, 0)
    wb_drain(NSB - 2, accs[0], semws[0])
    wb_drain(NSB - 1, accs[1], semws[1])


_sc_pool = functools.partial(
    pl.kernel,
    out_type=jax.ShapeDtypeStruct((N * G * H,), jnp.float32),
    mesh=plsc.VectorSubcoreMesh(core_axis_name="c", subcore_axis_name="s"),
    scratch_types=[
        pltpu.VMEM((JC * H,), jnp.float32),
        pltpu.VMEM((JC + LANES,), jnp.float32),
        pltpu.VMEM((JC + LANES,), jnp.float32),
        pltpu.VMEM((IPW,), jnp.int32),
        pltpu.VMEM((IPW,), jnp.int32),
        pltpu.VMEM((IPW,), jnp.float32),
        pltpu.VMEM((IPW,), jnp.float32),
        pltpu.VMEM((ACC + SB * H,), jnp.float32),
        pltpu.VMEM((ACC + SB * H,), jnp.float32),
        pltpu.SemaphoreType.DMA,
        pltpu.SemaphoreType.DMA,
        pltpu.SemaphoreType.DMA,
    ],
)(_sc_body)


def kernel(ht, pos_t, same_scene_mask):
    ht2 = ht.reshape(N, H)
    pos = pos_t.reshape(N, 2)
    scene = same_scene_mask.reshape(N)
    idx = jnp.arange(N, dtype=jnp.int32)
    prev_ne = jnp.concatenate(
        [jnp.ones((1,), bool), scene[1:] != scene[:-1]])
    next_ne = jnp.concatenate(
        [scene[1:] != scene[:-1], jnp.ones((1,), bool)])
    rs = lax.associative_scan(jnp.maximum, jnp.where(prev_ne, idx, 0))
    re_ = lax.associative_scan(jnp.minimum, jnp.where(next_ne, idx + 1, N),
                               reverse=True)
    zf = jnp.zeros((JC,), jnp.float32)
    ht_pad = jnp.concatenate(
        [ht2, jnp.zeros((JC, H), ht2.dtype)], axis=0).reshape((N + JC) * H)
    px_pad = jnp.concatenate([pos[:, 0], zf])
    py_pad = jnp.concatenate([pos[:, 1], zf])
    out = _sc_pool(ht_pad, px_pad, py_pad, rs, re_)
    return out.reshape(N, G, H)
